# Initial kernel scaffold; baseline (speedup 1.0000x reference)
import jax
import jax.numpy as jnp
from jax.experimental import pallas as pl


def kernel(x, fc_weight, bias):
    raise NotImplementedError("write your pallas kernel here")


# speedup vs baseline: 1.4937x; 1.4937x over previous
import functools

import jax
import jax.numpy as jnp
from jax import lax
from jax.experimental import pallas as pl
from jax.experimental.pallas import tpu as pltpu
from jax.experimental.pallas import tpu_sc as plsc

_FIELD_DIM = 38462
_F = 26
_B = 16384
_NC = 2
_NS = 16
_NW = _NC * _NS
_BW = _B // _NW
_L = 16
_CHUNK = 128
_QPF = _BW // _CHUNK

_mesh = plsc.VectorSubcoreMesh(core_axis_name="c", subcore_axis_name="s")


@functools.partial(
    pl.kernel,
    mesh=_mesh,
    out_type=jax.ShapeDtypeStruct((_B,), jnp.float32),
    scratch_types=[
        pltpu.VMEM((_F * _BW,), jnp.int32),
        pltpu.VMEM((_F * _BW,), jnp.float32),
        pltpu.VMEM((_BW,), jnp.float32),
        pltpu.VMEM((_L,), jnp.float32),
        pltpu.SemaphoreType.DMA,
    ],
)
def _embed_sum(xT, wt, bias, out, idx_v, g_v, o_v, bias_v, sem):
    c = lax.axis_index("c")
    s = lax.axis_index("s")
    wid = s * _NC + c
    base = wid * _BW

    pltpu.sync_copy(bias.at[pl.ds(0, 1)], bias_v.at[pl.ds(0, 1)])

    def per_field(f, _):
        row = pl.ds(f * _BW, _BW)
        pltpu.sync_copy(xT.at[f, pl.ds(base, _BW)], idx_v.at[row])
        off = f * _FIELD_DIM

        def add16(j, _):
            sl = pl.ds(f * _BW + j * _L, _L)
            idx_v[sl] = idx_v[sl] + off
            return 0

        lax.fori_loop(0, _BW // _L, add16, 0)

        def fire(q, _):
            qs = pl.ds(f * _BW + q * _CHUNK, _CHUNK)
            pltpu.make_async_copy(wt.at[idx_v.at[qs]], g_v.at[qs], sem).start()
            return 0

        lax.fori_loop(0, _QPF, fire, 0)
        return 0

    lax.fori_loop(0, _F, per_field, 0)

    pltpu.make_async_copy(wt.at[pl.ds(0, _F * _BW)], g_v, sem).wait()

    bias_s = bias_v[pl.ds(0, _L)][0]

    def reduce16(j, _):
        acc = jnp.zeros((_L,), jnp.float32) + bias_s
        for f in range(_F):
            acc = acc + g_v[pl.ds(f * _BW + j * _L, _L)]
        o_v[pl.ds(j * _L, _L)] = acc
        return 0

    lax.fori_loop(0, _BW // _L, reduce16, 0)

    pltpu.sync_copy(o_v, out.at[pl.ds(base, _BW)])


def kernel(x, fc_weight, bias):
    xT = x.T
    wt = fc_weight.reshape(-1)
    y = _embed_sum(xT, wt, bias)
    return y.reshape(_B, 1)
